# baseline (speedup 1.0000x reference)
import functools

import jax
import jax.numpy as jnp
from jax import lax
from jax.experimental import pallas as pl
from jax.experimental.pallas import tpu as pltpu
from jax.experimental.pallas import tpu_sc as plsc

N = 10000
E = 320000
NC = 2
NS = 16
NW = NC * NS
CHUNK = 128
NCHUNKS = E // CHUNK
CH_PER_W = NCHUNKS // NW
EXTRA_CHUNKS = NCHUNKS - CH_PER_W * NW
ROWS_PER_TILE = 624
EXTRA_ROWS = N - NS * ROWS_PER_TILE


def _make_seg_sum(F):
    mesh = plsc.VectorSubcoreMesh(core_axis_name="c", subcore_axis_name="s")

    @functools.partial(
        pl.kernel,
        mesh=mesh,
        compiler_params=pltpu.CompilerParams(use_tc_tiling_on_sc=False),
        out_type=(jax.ShapeDtypeStruct((N, F), jnp.float32),
                  jax.ShapeDtypeStruct((N, F), jnp.float32)),
        scratch_types=[
            pltpu.VMEM((CH_PER_W * CHUNK,), jnp.int32),
            pltpu.VMEM((2, CHUNK), jnp.int32),
            pltpu.VMEM((CHUNK, F), jnp.float32),
            pltpu.VMEM((CHUNK, F), jnp.float32),
            pltpu.VMEM_SHARED((N, F), jnp.float32),
            pltpu.SemaphoreType.DMA,
            pltpu.SemaphoreType.DMA,
            pltpu.SemaphoreType.DMA,
            pltpu.SemaphoreType.DMA,
        ],
    )
    def seg_sum(h_hbm, src_hbm, dst2d_hbm, out0_hbm, out1_hbm,
                idx_s, idx_d, rows0, rows1, acc,
                semg0, semg1, semd0, semd1):
        c = lax.axis_index("c")
        s = lax.axis_index("s")
        wid = s * NC + c
        cbase = wid * CH_PER_W
        rbase = s * ROWS_PER_TILE

        nfullcopy = ROWS_PER_TILE // CHUNK
        rem = ROWS_PER_TILE - nfullcopy * CHUNK

        @pl.when(c == 0)
        def _init_h():
            pltpu.sync_copy(h_hbm.at[pl.ds(rbase, ROWS_PER_TILE)],
                            acc.at[pl.ds(rbase, ROWS_PER_TILE)])

            @pl.when(s == NS - 1)
            def _init_h_extra():
                pltpu.sync_copy(h_hbm.at[pl.ds(NS * ROWS_PER_TILE, EXTRA_ROWS)],
                                acc.at[pl.ds(NS * ROWS_PER_TILE, EXTRA_ROWS)])

        @pl.when(c == 1)
        def _init_zero():
            zeros16 = jnp.zeros((16,), jnp.float32)
            fvec = F // 16

            @pl.loop(0, CHUNK * fvec, unroll=8)
            def _zero(i):
                rows0[i // fvec, pl.ds((i % fvec) * 16, 16)] = zeros16

            for j in range(nfullcopy):
                pltpu.sync_copy(rows0, acc.at[pl.ds(rbase + j * CHUNK, CHUNK)])
            if rem:
                pltpu.sync_copy(rows0.at[pl.ds(0, rem)],
                                acc.at[pl.ds(rbase + nfullcopy * CHUNK, rem)])

            @pl.when(s == NS - 1)
            def _zero_extra():
                pltpu.sync_copy(rows0.at[pl.ds(0, EXTRA_ROWS)],
                                acc.at[pl.ds(NS * ROWS_PER_TILE, EXTRA_ROWS)])

        pltpu.sync_copy(src_hbm.at[pl.ds(cbase * CHUNK, CH_PER_W * CHUNK)],
                        idx_s)

        rows = (rows0, rows1)
        semg = (semg0, semg1)
        semd = (semd0, semd1)
        pltpu.async_copy(h_hbm.at[idx_s.at[pl.ds(0, CHUNK)]], rows0, semg0)
        pltpu.async_copy(dst2d_hbm.at[pl.ds(cbase, 1)],
                         idx_d.at[pl.ds(0, 1)], semd0)
        plsc.subcore_barrier()

        @pl.loop(0, CH_PER_W, step=2)
        def _go(i):
            for b in range(2):
                cc = i + b
                nxt = 1 - b

                def _issue(nc=cc + 1, nb=nxt):
                    pltpu.async_copy(
                        h_hbm.at[idx_s.at[pl.ds(nc * CHUNK, CHUNK)]],
                        rows[nb], semg[nb])
                    pltpu.async_copy(dst2d_hbm.at[pl.ds(cbase + nc, 1)],
                                     idx_d.at[pl.ds(nb, 1)], semd[nb])

                if b == 0:
                    _issue()
                else:
                    pl.when(i < CH_PER_W - 2)(_issue)
                pltpu.make_async_copy(h_hbm.at[idx_s.at[pl.ds(0, CHUNK)]],
                                      rows[b], semg[b]).wait()
                pltpu.make_async_copy(dst2d_hbm.at[pl.ds(0, 1)],
                                      idx_d.at[pl.ds(b, 1)], semd[b]).wait()
                pltpu.sync_copy(rows[b], acc.at[idx_d.at[b]], add=True)

        @pl.when(wid < EXTRA_CHUNKS)
        def _extra():
            ck = NW * CH_PER_W + wid
            pltpu.sync_copy(src_hbm.at[pl.ds(ck * CHUNK, CHUNK)],
                            idx_s.at[pl.ds(0, CHUNK)])
            pltpu.sync_copy(dst2d_hbm.at[pl.ds(ck, 1)], idx_d.at[pl.ds(0, 1)])
            pltpu.async_copy(h_hbm.at[idx_s.at[pl.ds(0, CHUNK)]], rows0,
                             semg0).wait()
            pltpu.sync_copy(rows0, acc.at[idx_d.at[0]], add=True)

        plsc.subcore_barrier()

        for cc_, out_ in ((0, out0_hbm), (1, out1_hbm)):
            @pl.when(c == cc_)
            def _write(out_=out_):
                pltpu.sync_copy(acc.at[pl.ds(rbase, ROWS_PER_TILE)],
                                out_.at[pl.ds(rbase, ROWS_PER_TILE)])

                @pl.when(s == NS - 1)
                def _write_extra():
                    pltpu.sync_copy(
                        acc.at[pl.ds(NS * ROWS_PER_TILE, EXTRA_ROWS)],
                        out_.at[pl.ds(NS * ROWS_PER_TILE, EXTRA_ROWS)])

    return seg_sum


def _make_mlp(Fin, R=1000):
    H = 64
    NBLK = N // R

    def body(g0_hbm, g1_hbm, w1_ref, b1_ref, w2_ref, b2_ref, o_hbm,
             a0b0, a0b1, a1b0, a1b1, ob0, ob1,
             sa00, sa01, sa10, sa11, so0, so1):
        k = pl.program_id(0)
        a0bufs = (a0b0, a0b1)
        a1bufs = (a1b0, a1b1)
        obufs = (ob0, ob1)
        sa0 = (sa00, sa01)
        sa1 = (sa10, sa11)
        so = (so0, so1)

        def issue_load(blk, slot):
            pltpu.async_copy(g0_hbm.at[pl.ds(blk * R, R)],
                             a0bufs[slot], sa0[slot])
            pltpu.async_copy(g1_hbm.at[pl.ds(blk * R, R)],
                             a1bufs[slot], sa1[slot])

        @pl.when(k == 0)
        def _prologue():
            issue_load(0, 0)
            issue_load(1, 1)

        for slot in range(2):
            blk = 2 * k + slot
            pltpu.make_async_copy(g0_hbm.at[pl.ds(0, R)],
                                  a0bufs[slot], sa0[slot]).wait()
            pltpu.make_async_copy(g1_hbm.at[pl.ds(0, R)],
                                  a1bufs[slot], sa1[slot]).wait()
            u = a0bufs[slot][...] + a1bufs[slot][...]
            z = jnp.dot(u, w1_ref[...], preferred_element_type=jnp.float32)
            z = jnp.maximum(z + b1_ref[...], 0.0)
            o = jnp.dot(z, w2_ref[...], preferred_element_type=jnp.float32)
            o = jnp.maximum(o + b2_ref[...], 0.0)

            @pl.when(k > 0)
            def _drain_prev_out():
                pltpu.make_async_copy(obufs[slot], o_hbm.at[pl.ds(0, R)],
                                      so[slot]).wait()

            obufs[slot][...] = o
            pltpu.async_copy(obufs[slot], o_hbm.at[pl.ds(blk * R, R)],
                             so[slot])

            @pl.when(k < (NBLK // 2) - 1)
            def _prefetch(slot=slot):
                nblk = 2 * (k + 1) + slot
                pltpu.async_copy(g0_hbm.at[pl.ds(nblk * R, R)],
                                 a0bufs[slot], sa0[slot])
                pltpu.async_copy(g1_hbm.at[pl.ds(nblk * R, R)],
                                 a1bufs[slot], sa1[slot])

        @pl.when(k == (NBLK // 2) - 1)
        def _drain_final_outs():
            for slot in range(2):
                pltpu.make_async_copy(obufs[slot], o_hbm.at[pl.ds(0, R)],
                                      so[slot]).wait()

    grid = (NBLK // 2,)
    return pl.pallas_call(
        body,
        grid=grid,
        in_specs=[
            pl.BlockSpec(memory_space=pltpu.HBM),
            pl.BlockSpec(memory_space=pltpu.HBM),
            pl.BlockSpec((Fin, H), lambda i: (0, 0)),
            pl.BlockSpec((1, H), lambda i: (0, 0)),
            pl.BlockSpec((H, H), lambda i: (0, 0)),
            pl.BlockSpec((1, H), lambda i: (0, 0)),
        ],
        out_specs=pl.BlockSpec(memory_space=pltpu.HBM),
        out_shape=jax.ShapeDtypeStruct((N, H), jnp.float32),
        scratch_shapes=[
            pltpu.VMEM((R, Fin), jnp.float32),
            pltpu.VMEM((R, Fin), jnp.float32),
            pltpu.VMEM((R, Fin), jnp.float32),
            pltpu.VMEM((R, Fin), jnp.float32),
            pltpu.VMEM((R, H), jnp.float32),
            pltpu.VMEM((R, H), jnp.float32),
            pltpu.SemaphoreType.DMA,
            pltpu.SemaphoreType.DMA,
            pltpu.SemaphoreType.DMA,
            pltpu.SemaphoreType.DMA,
            pltpu.SemaphoreType.DMA,
            pltpu.SemaphoreType.DMA,
        ],
    )


def kernel(x, edge_index, W1_0, b1_0, W2_0, b2_0, W1_1, b1_1, W2_1, b2_1,
           W1_2, b1_2, W2_2, b2_2):
    src = edge_index[0]
    dst2d = edge_index[1].reshape(NCHUNKS, CHUNK)
    params = [(W1_0, b1_0, W2_0, b2_0), (W1_1, b1_1, W2_1, b2_1),
              (W1_2, b1_2, W2_2, b2_2)]
    h = x
    outs = []
    for (W1, b1, W2, b2) in params:
        F = h.shape[1]
        g0, g1 = _make_seg_sum(F)(h, src, dst2d)
        h = _make_mlp(F)(g0, g1, W1, b1.reshape(1, -1), W2, b2.reshape(1, -1))
        outs.append(h)
    return jnp.concatenate(outs, axis=1)
